# baseline (speedup 1.0000x reference)
import functools

import jax
import jax.numpy as jnp
from jax import lax
from jax.experimental import pallas as pl
from jax.experimental.pallas import tpu as pltpu
from jax.experimental.pallas import tpu_sc as plsc

F = 26
V = 100000
D = 64
B = 4096
NC = 2
NS = 16
NW = NC * NS
BPW = B // NW
CHUNK = 16
NCHUNK = BPW // CHUNK
ROWS_PER_CHUNK = CHUNK * F
GATHER_MINOR = 104
GATHERS_PER_CHUNK = ROWS_PER_CHUNK // GATHER_MINOR

_mesh = plsc.VectorSubcoreMesh(core_axis_name="c", subcore_axis_name="s")


@functools.partial(
    pl.kernel,
    mesh=_mesh,
    compiler_params=pltpu.CompilerParams(
        needs_layout_passes=False, use_tc_tiling_on_sc=False),
    out_type=jax.ShapeDtypeStruct((B,), jnp.float32),
    scratch_types=[
        pltpu.VMEM((BPW * F // GATHER_MINOR, GATHER_MINOR), jnp.int32),
        pltpu.VMEM((ROWS_PER_CHUNK, D), jnp.float32),
        pltpu.VMEM((ROWS_PER_CHUNK, D), jnp.float32),
        pltpu.VMEM((ROWS_PER_CHUNK,), jnp.float32),
        pltpu.VMEM((ROWS_PER_CHUNK,), jnp.float32),
        pltpu.VMEM((16,), jnp.float32),
        pltpu.VMEM((BPW,), jnp.float32),
        pltpu.SemaphoreType.DMA,
        pltpu.SemaphoreType.DMA,
    ],
)
def _fm_sc(emb_hbm, idx_hbm, lin_hbm, bias_hbm, out_hbm,
           idx_v, rows0, rows1, lin0, lin1, bias_v, out_v, sem0, sem1):
    wid = lax.axis_index("s") * NC + lax.axis_index("c")
    base = wid * BPW

    pltpu.sync_copy(idx_hbm.at[wid], idx_v)
    pltpu.sync_copy(bias_hbm, bias_v)
    bias_vec = bias_v[...]

    rows = (rows0, rows1)
    lins = (lin0, lin1)
    sems = (sem0, sem1)

    def start(c):
        bi = c % 2
        hs = []
        for j in range(GATHERS_PER_CHUNK):
            g = c * GATHERS_PER_CHUNK + j
            hs.append(pltpu.async_copy(
                emb_hbm.at[idx_v.at[g]],
                rows[bi].at[pl.ds(j * GATHER_MINOR, GATHER_MINOR)],
                sems[bi]))
            hs.append(pltpu.async_copy(
                lin_hbm.at[idx_v.at[g]],
                lins[bi].at[pl.ds(j * GATHER_MINOR, GATHER_MINOR)],
                sems[bi]))
        return hs

    lanes = lax.iota(jnp.int32, 16)
    row_ids = [lanes * F + f for f in range(F)]
    zero = jnp.zeros((16,), jnp.float32)

    def compute(c):
        bi = c % 2
        rbuf, lbuf = rows[bi], lins[bi]

        def body_d(d, carry):
            acc_ss, acc_t = carry
            dcol = jnp.full((16,), d, dtype=jnp.int32)
            s = zero
            t = acc_t
            for f in range(F):
                e = plsc.load_gather(rbuf, [row_ids[f], dcol])
                s = s + e
                t = t + e * e
            return (acc_ss + s * s, t)

        acc_ss, acc_t = lax.fori_loop(0, D, body_d, (zero, zero))
        lin_sum = zero
        for f in range(F):
            lin_sum = lin_sum + plsc.load_gather(lbuf, [row_ids[f]])
        res = bias_vec + lin_sum + 0.5 * (acc_ss - acc_t)
        out_v[pl.ds(c * CHUNK, CHUNK)] = 1.0 / (1.0 + jnp.exp(-res))

    pending = start(0)
    for c in range(NCHUNK):
        nxt = start(c + 1) if c + 1 < NCHUNK else []
        for h in pending:
            h.wait()
        compute(c)
        pending = nxt

    pltpu.sync_copy(out_v, out_hbm.at[pl.ds(base, BPW)])


def kernel(x, emb_tables, lin_tables, bias):
    idx = (x.astype(jnp.int32)
           + (jnp.arange(F, dtype=jnp.int32) * V)[None, :])
    idx = idx.reshape(NW, BPW * F // GATHER_MINOR, GATHER_MINOR)
    emb_flat = emb_tables.reshape(F * V, D)
    lin_flat = lin_tables.reshape(F * V)
    bias16 = jnp.broadcast_to(bias, (16,))
    out = _fm_sc(emb_flat, idx, lin_flat, bias16)
    return out.reshape(B, 1)


# speedup vs baseline: 3.1910x; 3.1910x over previous
import functools

import jax
import jax.numpy as jnp
from jax import lax
from jax.experimental import pallas as pl
from jax.experimental.pallas import tpu as pltpu
from jax.experimental.pallas import tpu_sc as plsc

F = 26
V = 100000
D = 64
B = 4096
NC = 2
NS = 16
NW = NC * NS
DPW = D // NW
NG = B // 16

_mesh = plsc.VectorSubcoreMesh(core_axis_name="c", subcore_axis_name="s")
_params = pltpu.CompilerParams(needs_layout_passes=False)


@functools.partial(
    pl.kernel,
    mesh=_mesh,
    compiler_params=_params,
    out_type=jax.ShapeDtypeStruct((NW, B), jnp.float32),
    scratch_types=[
        pltpu.VMEM((V,), jnp.float32),
        pltpu.VMEM((B,), jnp.int32),
        pltpu.VMEM((B,), jnp.float32),
        pltpu.VMEM((B,), jnp.float32),
        pltpu.VMEM((B,), jnp.float32),
        pltpu.VMEM((B,), jnp.float32),
    ],
)
def _fm_part(emb_hbm, xt_hbm, lin_hbm, out_hbm,
             row_v, xidx, s0, s1, t, part):
    w = lax.axis_index("s") * NC + lax.axis_index("c")
    d0 = w * DPW

    zero = jnp.zeros((16,), jnp.float32)

    def zero_body(g, _):
        sl = pl.ds(g * 16, 16)
        s0[sl] = zero
        s1[sl] = zero
        t[sl] = zero
        return 0

    lax.fori_loop(0, NG, zero_body, 0)

    def accum_into(s_ref):
        def body(g, _):
            sl = pl.ds(g * 16, 16)
            e = plsc.load_gather(row_v, [xidx[sl]])
            s_ref[sl] = s_ref[sl] + e
            t[sl] = t[sl] + e * e
            return 0
        lax.fori_loop(0, NG, body, 0)

    s_refs = (s0, s1)
    for f in range(F):
        pltpu.sync_copy(xt_hbm.at[f], xidx)
        for dj in range(DPW):
            pltpu.sync_copy(emb_hbm.at[f * D + d0 + dj], row_v)
            accum_into(s_refs[dj])

    def fm_body(g, _):
        sl = pl.ds(g * 16, 16)
        a, b_, c = s0[sl], s1[sl], t[sl]
        part[sl] = 0.5 * (a * a + b_ * b_ - c)
        return 0

    lax.fori_loop(0, NG, fm_body, 0)

    @pl.when(w < F)
    def _():
        pltpu.sync_copy(xt_hbm.at[w], xidx)
        pltpu.sync_copy(lin_hbm.at[pl.ds(w * V, V)], row_v)

        def lin_body(g, _):
            sl = pl.ds(g * 16, 16)
            part[sl] = part[sl] + plsc.load_gather(row_v, [xidx[sl]])
            return 0

        lax.fori_loop(0, NG, lin_body, 0)

    pltpu.sync_copy(part, out_hbm.at[w])


@functools.partial(
    pl.kernel,
    mesh=_mesh,
    compiler_params=_params,
    out_type=jax.ShapeDtypeStruct((B,), jnp.float32),
    scratch_types=[
        pltpu.VMEM((NW, B // NW), jnp.float32),
        pltpu.VMEM((16,), jnp.float32),
        pltpu.VMEM((B // NW,), jnp.float32),
    ],
)
def _fm_combine(parts_hbm, bias_hbm, out_hbm, pbuf, bias_v, obuf):
    w = lax.axis_index("s") * NC + lax.axis_index("c")
    bpw = B // NW
    base = w * bpw
    pltpu.sync_copy(bias_hbm, bias_v)
    pltpu.sync_copy(parts_hbm.at[:, pl.ds(base, bpw)], pbuf)
    bias_vec = bias_v[...]

    def body(g, _):
        acc = bias_vec
        for u in range(NW):
            acc = acc + pbuf[u, pl.ds(g * 16, 16)]
        obuf[pl.ds(g * 16, 16)] = 1.0 / (1.0 + jnp.exp(-acc))
        return 0

    lax.fori_loop(0, bpw // 16, body, 0)
    pltpu.sync_copy(obuf, out_hbm.at[pl.ds(base, bpw)])


def kernel(x, emb_tables, lin_tables, bias):
    emb_t = jnp.swapaxes(emb_tables, 1, 2).reshape(F * D, V)
    xt = x.T.astype(jnp.int32)
    lin_flat = lin_tables.reshape(F * V)
    bias16 = jnp.broadcast_to(bias, (16,))
    parts = _fm_part(emb_t, xt, lin_flat)
    out = _fm_combine(parts, bias16)
    return out.reshape(B, 1)
